# Initial kernel scaffold; baseline (speedup 1.0000x reference)
import jax
import jax.numpy as jnp
from jax.experimental import pallas as pl


def kernel(z, edge_index, W1, b1, W2, b2, bias):
    raise NotImplementedError("write your pallas kernel here")


# speedup vs baseline: 25.6821x; 25.6821x over previous
import functools

import jax
import jax.numpy as jnp
from jax import lax
from jax.experimental import pallas as pl
from jax.experimental.pallas import tpu as pltpu
from jax.experimental.pallas import tpu_sc as plsc

N = 768
D = 128
E_Q = 49152

_SC_INFO = plsc.get_sparse_core_info()
_NC = _SC_INFO.num_cores
_NS = _SC_INFO.num_subcores
_NW = _NC * _NS
_EPW = E_Q // _NW
_GROUPS = _EPW // 16


def _dense_body(z_ref, w1_ref, b1_ref, w2_ref, b2_ref, h2_ref):
    z = z_ref[...]
    xw1 = jnp.dot(z, w1_ref[...], preferred_element_type=jnp.float32)
    m1 = jnp.sum(xw1, axis=0, keepdims=True) * (1.0 / N)
    h = jnp.maximum(z + m1 + b1_ref[...], 0.0)
    xw2 = jnp.dot(h, w2_ref[...], preferred_element_type=jnp.float32)
    m2 = jnp.sum(xw2, axis=0, keepdims=True) * (1.0 / N)
    h2_ref[...] = h + m2 + b2_ref[...]


def _dense(z, W1, b1, W2, b2):
    return pl.pallas_call(
        _dense_body,
        out_shape=jax.ShapeDtypeStruct((N, D), jnp.float32),
    )(z, W1, b1.reshape(1, D), W2, b2.reshape(1, D))


def _score_body(h2_hbm, src_hbm, dst_hbm, bias_hbm, out_hbm,
                h2_v, i_v, j_v, bias_v, out_v):
    wid = lax.axis_index("s") * _NC + lax.axis_index("c")
    base = wid * _EPW
    pltpu.sync_copy(h2_hbm, h2_v)
    pltpu.sync_copy(src_hbm.at[pl.ds(base, _EPW)], i_v)
    pltpu.sync_copy(dst_hbm.at[pl.ds(base, _EPW)], j_v)
    pltpu.sync_copy(bias_hbm, bias_v)
    bias = bias_v[...]

    def group(g, carry):
        ibase = i_v[pl.ds(g * 16, 16)] * D
        jbase = j_v[pl.ds(g * 16, 16)] * D

        def dstep(d, acc):
            a = plsc.load_gather(h2_v, [ibase + d])
            b = plsc.load_gather(h2_v, [jbase + d])
            return acc + a * b

        logit = lax.fori_loop(0, D, dstep, jnp.zeros((16,), jnp.float32)) + bias
        out_v[pl.ds(g * 16, 16)] = 1.0 / (1.0 + jnp.exp(-logit))
        return carry

    lax.fori_loop(0, _GROUPS, group, 0)
    pltpu.sync_copy(out_v, out_hbm.at[pl.ds(base, _EPW)])


@functools.partial(jax.jit, static_argnames=())
def _score(h2, src, dst, bias16):
    mesh = plsc.VectorSubcoreMesh(core_axis_name="c", subcore_axis_name="s")
    return pl.kernel(
        _score_body,
        out_type=jax.ShapeDtypeStruct((E_Q,), jnp.float32),
        mesh=mesh,
        compiler_params=pltpu.CompilerParams(
            use_tc_tiling_on_sc=False, needs_layout_passes=False),
        scratch_types=[
            pltpu.VMEM((N * D,), jnp.float32),
            pltpu.VMEM((_EPW,), jnp.int32),
            pltpu.VMEM((_EPW,), jnp.int32),
            pltpu.VMEM((16,), jnp.float32),
            pltpu.VMEM((_EPW,), jnp.float32),
        ],
    )(h2, src, dst, bias16)


def kernel(z, edge_index, W1, b1, W2, b2, bias):
    h2 = _dense(z, W1, b1, W2, b2)
    bias16 = jnp.broadcast_to(bias.astype(jnp.float32), (16,))
    return _score(h2.reshape(N * D), edge_index[0], edge_index[1], bias16)


# speedup vs baseline: 35.4985x; 1.3822x over previous
import functools

import jax
import jax.numpy as jnp
from jax import lax
from jax.experimental import pallas as pl
from jax.experimental.pallas import tpu as pltpu
from jax.experimental.pallas import tpu_sc as plsc

N = 768
D = 128
E_Q = 49152

_SC_INFO = plsc.get_sparse_core_info()
_NC = _SC_INFO.num_cores
_NS = _SC_INFO.num_subcores
_NW = _NC * _NS
_EPW = E_Q // _NW
_GROUPS = _EPW // 16


def _dense_body(z_ref, w1_ref, b1_ref, w2_ref, b2_ref, h2_ref):
    z = z_ref[...]
    xw1 = jnp.dot(z, w1_ref[...], preferred_element_type=jnp.float32)
    m1 = jnp.sum(xw1, axis=0, keepdims=True) * (1.0 / N)
    h = jnp.maximum(z + m1 + b1_ref[...], 0.0)
    xw2 = jnp.dot(h, w2_ref[...], preferred_element_type=jnp.float32)
    m2 = jnp.sum(xw2, axis=0, keepdims=True) * (1.0 / N)
    h2_ref[...] = h + m2 + b2_ref[...]


def _dense(z, W1, b1, W2, b2):
    return pl.pallas_call(
        _dense_body,
        out_shape=jax.ShapeDtypeStruct((N, D), jnp.float32),
    )(z, W1, b1.reshape(1, D), W2, b2.reshape(1, D))


def _score_body(h2_hbm, src_hbm, dst_hbm, bias_hbm, out_hbm,
                h2_v, i_v, j_v, bias_v, out_v):
    wid = lax.axis_index("s") * _NC + lax.axis_index("c")
    base = wid * _EPW
    pltpu.sync_copy(h2_hbm, h2_v)
    pltpu.sync_copy(src_hbm.at[pl.ds(base, _EPW)], i_v)
    pltpu.sync_copy(dst_hbm.at[pl.ds(base, _EPW)], j_v)
    pltpu.sync_copy(bias_hbm, bias_v)
    bias = bias_v[...]

    def group(g, carry):
        ibase = i_v[pl.ds(g * 16, 16)] * D
        jbase = j_v[pl.ds(g * 16, 16)] * D
        acc0 = jnp.zeros((16,), jnp.float32)
        acc1 = jnp.zeros((16,), jnp.float32)
        for d in range(0, D, 2):
            a0 = plsc.load_gather(h2_v, [ibase + d])
            b0 = plsc.load_gather(h2_v, [jbase + d])
            a1 = plsc.load_gather(h2_v, [ibase + (d + 1)])
            b1 = plsc.load_gather(h2_v, [jbase + (d + 1)])
            acc0 = acc0 + a0 * b0
            acc1 = acc1 + a1 * b1
        logit = acc0 + acc1 + bias
        out_v[pl.ds(g * 16, 16)] = 1.0 / (1.0 + jnp.exp(-logit))
        return carry

    lax.fori_loop(0, _GROUPS, group, 0)
    pltpu.sync_copy(out_v, out_hbm.at[pl.ds(base, _EPW)])


@functools.partial(jax.jit, static_argnames=())
def _score(h2, src, dst, bias16):
    mesh = plsc.VectorSubcoreMesh(core_axis_name="c", subcore_axis_name="s")
    return pl.kernel(
        _score_body,
        out_type=jax.ShapeDtypeStruct((E_Q,), jnp.float32),
        mesh=mesh,
        compiler_params=pltpu.CompilerParams(
            use_tc_tiling_on_sc=False, needs_layout_passes=False),
        scratch_types=[
            pltpu.VMEM((N * D,), jnp.float32),
            pltpu.VMEM((_EPW,), jnp.int32),
            pltpu.VMEM((_EPW,), jnp.int32),
            pltpu.VMEM((16,), jnp.float32),
            pltpu.VMEM((_EPW,), jnp.float32),
        ],
    )(h2, src, dst, bias16)


def kernel(z, edge_index, W1, b1, W2, b2, bias):
    h2 = _dense(z, W1, b1, W2, b2)
    bias16 = jnp.broadcast_to(bias.astype(jnp.float32), (16,))
    return _score(h2.reshape(N * D), edge_index[0], edge_index[1], bias16)


# speedup vs baseline: 236.0199x; 6.6487x over previous
import functools

import jax
import jax.numpy as jnp
from jax import lax
from jax.experimental import pallas as pl
from jax.experimental.pallas import tpu as pltpu
from jax.experimental.pallas import tpu_sc as plsc

N = 768
D = 128
E_Q = 49152

_SC_INFO = plsc.get_sparse_core_info()
_NC = _SC_INFO.num_cores
_NS = _SC_INFO.num_subcores
_NW = _NC * _NS
_EPW = E_Q // _NW
_GROUPS = _EPW // 16
_CH = 128
_NCH = _EPW // _CH


def _dense_body(z_ref, w1_ref, b1_ref, w2_ref, b2_ref, g_ref):
    z = z_ref[...]
    xw1 = jnp.dot(z, w1_ref[...], preferred_element_type=jnp.float32)
    m1 = jnp.sum(xw1, axis=0, keepdims=True) * (1.0 / N)
    h = jnp.maximum(z + m1 + b1_ref[...], 0.0)
    xw2 = jnp.dot(h, w2_ref[...], preferred_element_type=jnp.float32)
    m2 = jnp.sum(xw2, axis=0, keepdims=True) * (1.0 / N)
    h2 = h + m2 + b2_ref[...]
    g_ref[...] = lax.dot_general(
        h2, h2, (((1,), (1,)), ((), ())), preferred_element_type=jnp.float32)


def _dense(z, W1, b1, W2, b2):
    return pl.pallas_call(
        _dense_body,
        out_shape=jax.ShapeDtypeStruct((N, N), jnp.float32),
    )(z, W1, b1.reshape(1, D), W2, b2.reshape(1, D))


def _score_body(g_hbm, src_hbm, dst_hbm, bias_hbm, out_hbm,
                i_v, j_v, fidx_v, gat_v, bias_v, sem):
    wid = lax.axis_index("s") * _NC + lax.axis_index("c")
    base = wid * _EPW
    pltpu.sync_copy(src_hbm.at[pl.ds(base, _EPW)], i_v)
    pltpu.sync_copy(dst_hbm.at[pl.ds(base, _EPW)], j_v)
    pltpu.sync_copy(bias_hbm, bias_v)
    bias = bias_v[...]

    def mkidx(g, carry):
        i16 = i_v[pl.ds(g * 16, 16)]
        j16 = j_v[pl.ds(g * 16, 16)]
        fidx_v[pl.ds(g * 16, 16)] = i16 * N + j16
        return carry

    lax.fori_loop(0, _GROUPS, mkidx, 0)

    copies = [
        pltpu.async_copy(g_hbm.at[fidx_v.at[pl.ds(c * _CH, _CH)]],
                         gat_v.at[pl.ds(c * _CH, _CH)], sem)
        for c in range(_NCH)
    ]
    for c in copies:
        c.wait()

    def act(g, carry):
        logit = gat_v[pl.ds(g * 16, 16)] + bias
        gat_v[pl.ds(g * 16, 16)] = 1.0 / (1.0 + jnp.exp(-logit))
        return carry

    lax.fori_loop(0, _GROUPS, act, 0)
    pltpu.sync_copy(gat_v, out_hbm.at[pl.ds(base, _EPW)])


@functools.partial(jax.jit, static_argnames=())
def _score(g_flat, src, dst, bias16):
    mesh = plsc.VectorSubcoreMesh(core_axis_name="c", subcore_axis_name="s")
    out = pl.kernel(
        _score_body,
        out_type=jax.ShapeDtypeStruct((E_Q,), jnp.float32),
        mesh=mesh,
        compiler_params=pltpu.CompilerParams(
            use_tc_tiling_on_sc=False, needs_layout_passes=False),
        scratch_types=[
            pltpu.VMEM((_EPW,), jnp.int32),
            pltpu.VMEM((_EPW,), jnp.int32),
            pltpu.VMEM((_EPW,), jnp.int32),
            pltpu.VMEM((_EPW,), jnp.float32),
            pltpu.VMEM((16,), jnp.float32),
            pltpu.SemaphoreType.DMA,
        ],
    )(g_flat, src, dst, bias16)
    return out


def kernel(z, edge_index, W1, b1, W2, b2, bias):
    g = _dense(z, W1, b1, W2, b2)
    bias16 = jnp.broadcast_to(bias.astype(jnp.float32), (16,))
    return _score(g.reshape(N * N), edge_index[0], edge_index[1], bias16)


# speedup vs baseline: 249.6574x; 1.0578x over previous
import functools

import jax
import jax.numpy as jnp
from jax import lax
from jax.experimental import pallas as pl
from jax.experimental.pallas import tpu as pltpu
from jax.experimental.pallas import tpu_sc as plsc

N = 768
D = 128
E_Q = 49152

_SC_INFO = plsc.get_sparse_core_info()
_NC = _SC_INFO.num_cores
_NS = _SC_INFO.num_subcores
_NW = _NC * _NS
_EPW = E_Q // _NW
_CH = 128
_NCH = _EPW // _CH
_GPC = _CH // 16


def _dense_body(z_ref, w1_ref, b1_ref, w2_ref, b2_ref, g_ref):
    z = z_ref[...]
    xw1 = jnp.dot(z, w1_ref[...], preferred_element_type=jnp.float32)
    m1 = jnp.sum(xw1, axis=0, keepdims=True) * (1.0 / N)
    h = jnp.maximum(z + m1 + b1_ref[...], 0.0)
    xw2 = jnp.dot(h, w2_ref[...], preferred_element_type=jnp.float32)
    m2 = jnp.sum(xw2, axis=0, keepdims=True) * (1.0 / N)
    h2 = h + m2 + b2_ref[...]
    g_ref[...] = lax.dot_general(
        h2, h2, (((1,), (1,)), ((), ())), preferred_element_type=jnp.float32)


def _dense(z, W1, b1, W2, b2):
    return pl.pallas_call(
        _dense_body,
        out_shape=jax.ShapeDtypeStruct((N, N), jnp.float32),
    )(z, W1, b1.reshape(1, D), W2, b2.reshape(1, D))


def _score_body(g_hbm, ei_hbm, bias_hbm, out_hbm,
                i_v, j_v, fidx_v, gat_v, bias_v, sem):
    wid = lax.axis_index("s") * _NC + lax.axis_index("c")
    base = wid * _EPW
    pltpu.sync_copy(ei_hbm.at[0, pl.ds(base, _EPW)], i_v)
    pltpu.sync_copy(ei_hbm.at[1, pl.ds(base, _EPW)], j_v)
    pltpu.sync_copy(bias_hbm, bias_v.at[pl.ds(0, 1)])

    copies = []
    for c in range(_NCH):
        for u in range(_GPC):
            o = c * _CH + u * 16
            fidx_v[pl.ds(o, 16)] = i_v[pl.ds(o, 16)] * N + j_v[pl.ds(o, 16)]
        copies.append(
            pltpu.async_copy(g_hbm.at[fidx_v.at[pl.ds(c * _CH, _CH)]],
                             gat_v.at[pl.ds(c * _CH, _CH)], sem))
    bias = bias_v[...][0]
    for c in copies:
        c.wait()
    for g in range(_EPW // 16):
        logit = gat_v[pl.ds(g * 16, 16)] + bias
        gat_v[pl.ds(g * 16, 16)] = 1.0 / (1.0 + jnp.exp(-logit))
    pltpu.sync_copy(gat_v, out_hbm.at[pl.ds(base, _EPW)])


@functools.partial(jax.jit, static_argnames=())
def _score(g_flat, edge_index, bias):
    mesh = plsc.VectorSubcoreMesh(core_axis_name="c", subcore_axis_name="s")
    out = pl.kernel(
        _score_body,
        out_type=jax.ShapeDtypeStruct((E_Q,), jnp.float32),
        mesh=mesh,
        compiler_params=pltpu.CompilerParams(
            use_tc_tiling_on_sc=False, needs_layout_passes=False),
        scratch_types=[
            pltpu.VMEM((_EPW,), jnp.int32),
            pltpu.VMEM((_EPW,), jnp.int32),
            pltpu.VMEM((_EPW,), jnp.int32),
            pltpu.VMEM((_EPW,), jnp.float32),
            pltpu.VMEM((16,), jnp.float32),
            pltpu.SemaphoreType.DMA,
        ],
    )(g_flat, edge_index, bias)
    return out


def kernel(z, edge_index, W1, b1, W2, b2, bias):
    g = _dense(z, W1, b1, W2, b2)
    return _score(g.reshape(N * N), edge_index, bias.astype(jnp.float32))


# speedup vs baseline: 257.7991x; 1.0326x over previous
import functools

import jax
import jax.numpy as jnp
from jax import lax
from jax.experimental import pallas as pl
from jax.experimental.pallas import tpu as pltpu
from jax.experimental.pallas import tpu_sc as plsc

N = 768
D = 128
E_Q = 49152

_SC_INFO = plsc.get_sparse_core_info()
_NC = _SC_INFO.num_cores
_NS = _SC_INFO.num_subcores
_NW = _NC * _NS
_EPW = E_Q // _NW
_CH = 128
_NCH = _EPW // _CH
_GPC = _CH // 16


def _dense_body(z_ref, w1_ref, b1_ref, w2_ref, b2_ref, g_ref):
    z = z_ref[...]
    xw1 = jnp.dot(z, w1_ref[...], preferred_element_type=jnp.float32)
    m1 = jnp.sum(xw1, axis=0, keepdims=True) * (1.0 / N)
    h = jnp.maximum(z + m1 + b1_ref[...], 0.0)
    xw2 = jnp.dot(h, w2_ref[...], preferred_element_type=jnp.float32)
    m2 = jnp.sum(xw2, axis=0, keepdims=True) * (1.0 / N)
    h2 = h + m2 + b2_ref[...]
    g_ref[...] = lax.dot_general(
        h2, h2, (((1,), (1,)), ((), ())), preferred_element_type=jnp.float32)


def _dense(z, W1, b1, W2, b2):
    return pl.pallas_call(
        _dense_body,
        out_shape=jax.ShapeDtypeStruct((N, N), jnp.float32),
    )(z, W1, b1.reshape(1, D), W2, b2.reshape(1, D))


def _score_body(g_hbm, ei_hbm, bias_hbm, out_hbm,
                i_v, j_v, fidx_v, gat_v, bias_v, sem):
    wid = lax.axis_index("s") * _NC + lax.axis_index("c")
    base = wid * _EPW
    in0 = pltpu.async_copy(ei_hbm.at[0, pl.ds(base, _EPW)], i_v, sem)
    in1 = pltpu.async_copy(ei_hbm.at[1, pl.ds(base, _EPW)], j_v, sem)
    in2 = pltpu.async_copy(bias_hbm, bias_v.at[pl.ds(0, 1)], sem)
    in0.wait()
    in1.wait()
    in2.wait()

    copies = []
    for c in range(_NCH):
        for u in range(_GPC):
            o = c * _CH + u * 16
            fidx_v[pl.ds(o, 16)] = i_v[pl.ds(o, 16)] * N + j_v[pl.ds(o, 16)]
        copies.append(
            pltpu.async_copy(g_hbm.at[fidx_v.at[pl.ds(c * _CH, _CH)]],
                             gat_v.at[pl.ds(c * _CH, _CH)], sem))
    bias = bias_v[...][0]
    for c in copies:
        c.wait()
    for g in range(_EPW // 16):
        logit = gat_v[pl.ds(g * 16, 16)] + bias
        gat_v[pl.ds(g * 16, 16)] = 1.0 / (1.0 + jnp.exp(-logit))
    pltpu.sync_copy(gat_v, out_hbm.at[pl.ds(base, _EPW)])


@functools.partial(jax.jit, static_argnames=())
def _score(g_flat, edge_index, bias):
    mesh = plsc.VectorSubcoreMesh(core_axis_name="c", subcore_axis_name="s")
    out = pl.kernel(
        _score_body,
        out_type=jax.ShapeDtypeStruct((E_Q,), jnp.float32),
        mesh=mesh,
        compiler_params=pltpu.CompilerParams(
            use_tc_tiling_on_sc=False, needs_layout_passes=False),
        scratch_types=[
            pltpu.VMEM((_EPW,), jnp.int32),
            pltpu.VMEM((_EPW,), jnp.int32),
            pltpu.VMEM((_EPW,), jnp.int32),
            pltpu.VMEM((_EPW,), jnp.float32),
            pltpu.VMEM((16,), jnp.float32),
            pltpu.SemaphoreType.DMA,
        ],
    )(g_flat, edge_index, bias)
    return out


def kernel(z, edge_index, W1, b1, W2, b2, bias):
    g = _dense(z, W1, b1, W2, b2)
    return _score(g.reshape(N * N), edge_index, bias.astype(jnp.float32))


# speedup vs baseline: 261.8174x; 1.0156x over previous
import functools

import jax
import jax.numpy as jnp
from jax import lax
from jax.experimental import pallas as pl
from jax.experimental.pallas import tpu as pltpu
from jax.experimental.pallas import tpu_sc as plsc

N = 768
D = 128
E_Q = 49152

_SC_INFO = plsc.get_sparse_core_info()
_NC = _SC_INFO.num_cores
_NS = _SC_INFO.num_subcores
_NW = _NC * _NS
_EPW = E_Q // _NW
_CH = 128
_NCH = _EPW // _CH


def _dense_body(z_ref, ei_ref, w1_ref, b1_ref, w2_ref, b2_ref, bias_ref,
                g_ref, fidx_ref):
    z = z_ref[...]
    xw1 = jnp.dot(z, w1_ref[...], preferred_element_type=jnp.float32)
    m1 = jnp.sum(xw1, axis=0, keepdims=True) * (1.0 / N)
    h = jnp.maximum(z + m1 + b1_ref[...], 0.0)
    xw2 = jnp.dot(h, w2_ref[...], preferred_element_type=jnp.float32)
    m2 = jnp.sum(xw2, axis=0, keepdims=True) * (1.0 / N)
    h2 = h + m2 + b2_ref[...]
    g_ref[...] = lax.dot_general(
        h2, h2, (((1,), (1,)), ((), ())),
        preferred_element_type=jnp.float32) + bias_ref[0, 0]
    ei = ei_ref[...]
    fidx_ref[...] = ei[0, :] * N + ei[1, :]


def _dense(z, edge_index, W1, b1, W2, b2, bias):
    return pl.pallas_call(
        _dense_body,
        out_shape=(
            jax.ShapeDtypeStruct((N, N), jnp.float32),
            jax.ShapeDtypeStruct((E_Q,), jnp.int32),
        ),
    )(z, edge_index, W1, b1.reshape(1, D), W2, b2.reshape(1, D),
      bias.reshape(1, 1).astype(jnp.float32))


def _score_body(g_hbm, fidx_hbm, out_hbm, fidx_v, gat_v, sem):
    wid = lax.axis_index("s") * _NC + lax.axis_index("c")
    base = wid * _EPW
    pltpu.sync_copy(fidx_hbm.at[pl.ds(base, _EPW)], fidx_v)
    copies = [
        pltpu.async_copy(g_hbm.at[fidx_v.at[pl.ds(c * _CH, _CH)]],
                         gat_v.at[pl.ds(c * _CH, _CH)], sem)
        for c in range(_NCH)
    ]
    for c in copies:
        c.wait()
    for g in range(_EPW // 16):
        logit = gat_v[pl.ds(g * 16, 16)]
        gat_v[pl.ds(g * 16, 16)] = 1.0 / (1.0 + jnp.exp(-logit))
    pltpu.sync_copy(gat_v, out_hbm.at[pl.ds(base, _EPW)])


@functools.partial(jax.jit, static_argnames=())
def _score(g_flat, fidx):
    mesh = plsc.VectorSubcoreMesh(core_axis_name="c", subcore_axis_name="s")
    out = pl.kernel(
        _score_body,
        out_type=jax.ShapeDtypeStruct((E_Q,), jnp.float32),
        mesh=mesh,
        compiler_params=pltpu.CompilerParams(
            use_tc_tiling_on_sc=False, needs_layout_passes=False),
        scratch_types=[
            pltpu.VMEM((_EPW,), jnp.int32),
            pltpu.VMEM((_EPW,), jnp.float32),
            pltpu.SemaphoreType.DMA,
        ],
    )(g_flat, fidx)
    return out


def kernel(z, edge_index, W1, b1, W2, b2, bias):
    g, fidx = _dense(z, edge_index, W1, b1, W2, b2, bias)
    return _score(g.reshape(N * N), fidx)


# speedup vs baseline: 262.0479x; 1.0009x over previous
import functools

import jax
import jax.numpy as jnp
from jax import lax
from jax.experimental import pallas as pl
from jax.experimental.pallas import tpu as pltpu
from jax.experimental.pallas import tpu_sc as plsc

N = 768
D = 128
E_Q = 49152

_SC_INFO = plsc.get_sparse_core_info()
_NC = _SC_INFO.num_cores
_NS = _SC_INFO.num_subcores
_NW = _NC * _NS
_EPW = E_Q // _NW
_CH = 128
_NCH = _EPW // _CH


def _dense_body(z_ref, ei_ref, w1_ref, b1_ref, w2_ref, b2_ref, bias_ref,
                g_ref, fidx_ref):
    z = z_ref[...]
    xw1 = jnp.dot(z, w1_ref[...], preferred_element_type=jnp.float32)
    m1 = jnp.sum(xw1, axis=0, keepdims=True) * (1.0 / N)
    h = jnp.maximum(z + m1 + b1_ref[...], 0.0)
    xw2 = jnp.dot(h, w2_ref[...], preferred_element_type=jnp.float32)
    m2 = jnp.sum(xw2, axis=0, keepdims=True) * (1.0 / N)
    h2 = h + m2 + b2_ref[...]
    g_ref[...] = -(lax.dot_general(
        h2, h2, (((1,), (1,)), ((), ())),
        preferred_element_type=jnp.float32) + bias_ref[0, 0])
    ei = ei_ref[...]
    fidx_ref[...] = ei[0, :] * N + ei[1, :]


def _dense(z, edge_index, W1, b1, W2, b2, bias):
    return pl.pallas_call(
        _dense_body,
        out_shape=(
            jax.ShapeDtypeStruct((N, N), jnp.float32),
            jax.ShapeDtypeStruct((E_Q,), jnp.int32),
        ),
    )(z, edge_index, W1, b1.reshape(1, D), W2, b2.reshape(1, D),
      bias.reshape(1, 1).astype(jnp.float32))


def _score_body(g_hbm, fidx_hbm, out_hbm, fidx_v, gat_v, res_v, sem):
    wid = lax.axis_index("s") * _NC + lax.axis_index("c")
    base = wid * _EPW
    pltpu.sync_copy(fidx_hbm.at[pl.ds(base, _EPW)], fidx_v)
    copies = [
        pltpu.async_copy(g_hbm.at[fidx_v.at[pl.ds(c * _CH, _CH)]],
                         gat_v.at[pl.ds(c * _CH, _CH)], sem)
        for c in range(_NCH)
    ]
    for c in range(_NCH):
        copies[c].wait()
        for u in range(_CH // 16):
            o = c * _CH + u * 16
            res_v[pl.ds(o, 16)] = 1.0 / (1.0 + jnp.exp(gat_v[pl.ds(o, 16)]))
    pltpu.sync_copy(res_v, out_hbm.at[pl.ds(base, _EPW)])


@functools.partial(jax.jit, static_argnames=())
def _score(g_flat, fidx):
    mesh = plsc.VectorSubcoreMesh(core_axis_name="c", subcore_axis_name="s")
    out = pl.kernel(
        _score_body,
        out_type=jax.ShapeDtypeStruct((E_Q,), jnp.float32),
        mesh=mesh,
        compiler_params=pltpu.CompilerParams(
            use_tc_tiling_on_sc=False, needs_layout_passes=False),
        scratch_types=[
            pltpu.VMEM((_EPW,), jnp.int32),
            pltpu.VMEM((_EPW,), jnp.float32),
            pltpu.VMEM((_EPW,), jnp.float32),
            pltpu.SemaphoreType.DMA,
        ],
    )(g_flat, fidx)
    return out


def kernel(z, edge_index, W1, b1, W2, b2, bias):
    g, fidx = _dense(z, edge_index, W1, b1, W2, b2, bias)
    return _score(g.reshape(N * N), fidx)


# speedup vs baseline: 283.2628x; 1.0810x over previous
import functools

import jax
import jax.numpy as jnp
from jax import lax
from jax.experimental import pallas as pl
from jax.experimental.pallas import tpu as pltpu
from jax.experimental.pallas import tpu_sc as plsc

N = 768
D = 128
E_Q = 49152

_SC_INFO = plsc.get_sparse_core_info()
_NC = _SC_INFO.num_cores
_NS = _SC_INFO.num_subcores
_NW = _NC * _NS
_EPW = E_Q // _NW
_CH = 128
_NCH = _EPW // _CH


def _dense_body(z_ref, ei_ref, w1_ref, b1_ref, w2_ref, b2_ref, bias_ref,
                g_ref, fidx_ref):
    z = z_ref[...]
    xw1 = jnp.dot(z, w1_ref[...], preferred_element_type=jnp.float32)
    m1 = jnp.sum(xw1, axis=0, keepdims=True) * (1.0 / N)
    h = jnp.maximum(z + m1 + b1_ref[...], 0.0)
    xw2 = jnp.dot(h, w2_ref[...], preferred_element_type=jnp.float32)
    m2 = jnp.sum(xw2, axis=0, keepdims=True) * (1.0 / N)
    h2 = h + m2 + b2_ref[...]
    gram = -(lax.dot_general(
        h2, h2, (((1,), (1,)), ((), ())),
        preferred_element_type=jnp.float32) + bias_ref[0, 0])
    g_ref[...] = gram.reshape(N * N // D, D)
    ei = ei_ref[...]
    fidx_ref[...] = ei[0, :] * N + ei[1, :]


def _dense(z, edge_index, W1, b1, W2, b2, bias):
    return pl.pallas_call(
        _dense_body,
        out_shape=(
            jax.ShapeDtypeStruct((N * N // D, D), jnp.float32),
            jax.ShapeDtypeStruct((E_Q,), jnp.int32),
        ),
    )(z, edge_index, W1, b1.reshape(1, D), W2, b2.reshape(1, D),
      bias.reshape(1, 1).astype(jnp.float32))


def _score_body(g_hbm, fidx_hbm, out_hbm, fidx_v, gat_v, res_v, sem):
    wid = lax.axis_index("s") * _NC + lax.axis_index("c")
    base = wid * _EPW
    pltpu.sync_copy(fidx_hbm.at[pl.ds(base, _EPW)], fidx_v)
    copies = [
        pltpu.async_copy(g_hbm.at[fidx_v.at[pl.ds(c * _CH, _CH)]],
                         gat_v.at[pl.ds(c * _CH, _CH)], sem)
        for c in range(_NCH)
    ]
    for c in range(_NCH):
        copies[c].wait()
        for u in range(_CH // 16):
            o = c * _CH + u * 16
            res_v[pl.ds(o, 16)] = 1.0 / (1.0 + jnp.exp(gat_v[pl.ds(o, 16)]))
    pltpu.sync_copy(res_v, out_hbm.at[pl.ds(base, _EPW)])


@functools.partial(jax.jit, static_argnames=())
def _score(g_flat, fidx):
    mesh = plsc.VectorSubcoreMesh(core_axis_name="c", subcore_axis_name="s")
    out = pl.kernel(
        _score_body,
        out_type=jax.ShapeDtypeStruct((E_Q,), jnp.float32),
        mesh=mesh,
        compiler_params=pltpu.CompilerParams(
            use_tc_tiling_on_sc=False, needs_layout_passes=False),
        scratch_types=[
            pltpu.VMEM((_EPW,), jnp.int32),
            pltpu.VMEM((_EPW,), jnp.float32),
            pltpu.VMEM((_EPW,), jnp.float32),
            pltpu.SemaphoreType.DMA,
        ],
    )(g_flat, fidx)
    return out


def kernel(z, edge_index, W1, b1, W2, b2, bias):
    g, fidx = _dense(z, edge_index, W1, b1, W2, b2, bias)
    return _score(g.reshape(N * N), fidx)
